# Initial kernel scaffold; baseline (speedup 1.0000x reference)
import functools

import jax
import jax.numpy as jnp
from jax import lax
from jax.experimental import pallas as pl
from jax.experimental.pallas import tpu as pltpu
from jax.experimental.pallas import tpu_sc as plsc

N = 10000
E = 320000
D = 128

NC = 2
NS = 16
NW = NC * NS

C = 128
CHUNKS = 80
EPW = C * CHUNKS
EP = EPW * NW

NPAD = 10240
ZROWS = NPAD // NS
OROWS = N // NS
DUMMY = N
DEGW = 16

R = 1000


def _sc_segment_sums(feat, srcp, dstp, zrows, zdeg, ones):
    mesh = plsc.VectorSubcoreMesh(core_axis_name="c", subcore_axis_name="s")

    @functools.partial(
        pl.kernel,
        out_type=(
            jax.ShapeDtypeStruct((NC, N, D), jnp.float32),
            jax.ShapeDtypeStruct((NC, N, DEGW), jnp.float32),
        ),
        mesh=mesh,
        scratch_types=[
            pltpu.VMEM_SHARED((NPAD, D), jnp.float32),
            pltpu.VMEM_SHARED((NPAD, DEGW), jnp.float32),
            pltpu.VMEM((CHUNKS, C), jnp.int32),
            pltpu.VMEM((CHUNKS, C), jnp.int32),
            pltpu.VMEM((2, C, D), jnp.float32),
            pltpu.VMEM((C, DEGW), jnp.float32),
            pltpu.SemaphoreType.DMA,
            pltpu.SemaphoreType.DMA,
        ],
    )
    def k(feat_hbm, srcp_hbm, dstp_hbm, zrows_hbm, zdeg_hbm, ones_hbm,
          nsum_hbm, deg_hbm, acc, dacc, srcl, dstl, rows, ones_v, sem0, sem1):
        c = lax.axis_index("c")
        s = lax.axis_index("s")
        w = c * NS + s

        pltpu.sync_copy(srcp_hbm.at[w], srcl)
        pltpu.sync_copy(dstp_hbm.at[w], dstl)
        pltpu.sync_copy(ones_hbm, ones_v)

        pltpu.sync_copy(zrows_hbm, acc.at[pl.ds(s * ZROWS, ZROWS)])
        pltpu.sync_copy(zdeg_hbm, dacc.at[pl.ds(s * ZROWS, ZROWS)])
        plsc.subcore_barrier()

        sems = (sem0, sem1)

        def gather(j, buf):
            pltpu.async_copy(feat_hbm.at[srcl.at[j]], rows.at[buf], sems[buf])

        def gwait(buf):
            pltpu.make_async_copy(
                feat_hbm.at[srcl.at[0]], rows.at[buf], sems[buf]).wait()

        def scatter(j, buf):
            pltpu.sync_copy(rows.at[buf], acc.at[dstl.at[j]], add=True)
            pltpu.sync_copy(ones_v, dacc.at[dstl.at[j]], add=True)

        gather(0, 0)

        @pl.loop(0, CHUNKS, step=2)
        def _(j):
            gather(j + 1, 1)
            gwait(0)
            scatter(j, 0)

            @pl.when(j + 2 < CHUNKS)
            def _():
                gather(j + 2, 0)

            gwait(1)
            scatter(j + 1, 1)

        plsc.subcore_barrier()

        pltpu.sync_copy(acc.at[pl.ds(s * OROWS, OROWS)],
                        nsum_hbm.at[c, pl.ds(s * OROWS, OROWS)])
        pltpu.sync_copy(dacc.at[pl.ds(s * OROWS, OROWS)],
                        deg_hbm.at[c, pl.ds(s * OROWS, OROWS)])

    return k(feat, srcp, dstp, zrows, zdeg, ones)


def _tc_combine(feat, nsum, deg, ws_t, wn_t):
    def body(feat_ref, n_ref, d_ref, ws_ref, wn_ref, o_ref):
        h = n_ref[0] + n_ref[1]
        dg = d_ref[0, :, :1] + d_ref[1, :, :1]
        h = h / jnp.maximum(dg, 1.0)
        o_ref[...] = (
            jnp.dot(feat_ref[...], ws_ref[...],
                    preferred_element_type=jnp.float32)
            + jnp.dot(h, wn_ref[...], preferred_element_type=jnp.float32))

    return pl.pallas_call(
        body,
        grid=(N // R,),
        in_specs=[
            pl.BlockSpec((R, D), lambda i: (i, 0)),
            pl.BlockSpec((NC, R, D), lambda i: (0, i, 0)),
            pl.BlockSpec((NC, R, DEGW), lambda i: (0, i, 0)),
            pl.BlockSpec((D, D), lambda i: (0, 0)),
            pl.BlockSpec((D, D), lambda i: (0, 0)),
        ],
        out_specs=pl.BlockSpec((R, D), lambda i: (i, 0)),
        out_shape=jax.ShapeDtypeStruct((N, D), jnp.float32),
    )(feat, nsum, deg, ws_t, wn_t)


def kernel(feat, edge_index, W_self, W_neigh):
    src = edge_index[0]
    dst = edge_index[1]
    pad = EP - E
    srcp = jnp.concatenate(
        [src, jnp.zeros((pad,), jnp.int32)]).reshape(NW, CHUNKS, C)
    dstp = jnp.concatenate(
        [dst, jnp.full((pad,), DUMMY, jnp.int32)]).reshape(NW, CHUNKS, C)
    zrows = jnp.zeros((ZROWS, D), jnp.float32)
    zdeg = jnp.zeros((ZROWS, DEGW), jnp.float32)
    ones = jnp.ones((C, DEGW), jnp.float32)
    nsum, deg = _sc_segment_sums(feat, srcp, dstp, zrows, zdeg, ones)
    return _tc_combine(feat, nsum, deg, W_self.T, W_neigh.T)


# speedup vs baseline: 3.8680x; 3.8680x over previous
import dataclasses
import functools

import jax
import jax.numpy as jnp
from jax import lax
from jax.experimental import pallas as pl
from jax.experimental.pallas import tpu as pltpu
from jax.experimental.pallas import tpu_sc as plsc

N = 10000
E = 320000
D = 128

NC = 2
NS = 16
NW = NC * NS
L = 16

C = 64
CHUNKS = 160
EPW = C * CHUNKS
EP = EPW * NW

NPAD = 10112
ZROWS = NPAD // NS
DUMMY = N
NDEG = 10240

R = 1000


def _sc_segment_sums(feat, srcf, dstf, zrows):
    mesh = plsc.VectorSubcoreMesh(core_axis_name="c", subcore_axis_name="s")
    cp = pltpu.CompilerParams()
    if "needs_layout_passes" in pltpu.CompilerParams.__dataclass_fields__:
        cp = dataclasses.replace(cp, needs_layout_passes=False)

    @functools.partial(
        pl.kernel,
        compiler_params=cp,
        out_type=(
            jax.ShapeDtypeStruct((NC, NPAD, D), jnp.float32),
            jax.ShapeDtypeStruct((NW * NDEG,), jnp.float32),
        ),
        mesh=mesh,
        scratch_types=[
            pltpu.VMEM_SHARED((NPAD, D), jnp.float32),
            pltpu.VMEM((C,), jnp.int32),
            pltpu.VMEM((C,), jnp.int32),
            pltpu.VMEM((C,), jnp.int32),
            pltpu.VMEM((C,), jnp.int32),
            pltpu.VMEM((C, D), jnp.float32),
            pltpu.VMEM((C, D), jnp.float32),
            pltpu.VMEM((NDEG,), jnp.float32),
            pltpu.SemaphoreType.DMA,
            pltpu.SemaphoreType.DMA,
            pltpu.SemaphoreType.DMA,
            pltpu.SemaphoreType.DMA,
        ],
    )
    def k(feat_hbm, srcf_hbm, dstf_hbm, zrows_hbm, nsum_hbm, deg_hbm,
          acc, sidx0, sidx1, didx0, didx1, rows0, rows1, degl,
          gsem0, gsem1, isem0, isem1):
        c = lax.axis_index("c")
        s = lax.axis_index("s")
        w = c * NS + s
        base = w * EPW

        @pl.loop(0, NDEG // L)
        def _(i):
            degl[pl.ds(i * L, L)] = jnp.zeros((L,), jnp.float32)

        pltpu.sync_copy(zrows_hbm, acc.at[pl.ds(s * ZROWS, ZROWS)])
        plsc.subcore_barrier()

        sidx = (sidx0, sidx1)
        didx = (didx0, didx1)
        rows = (rows0, rows1)
        isems = (isem0, isem1)
        ones_vec = jnp.full((L,), 1.0, jnp.float32)

        gath = (pltpu.make_async_copy(feat_hbm.at[sidx0], rows0, gsem0),
                pltpu.make_async_copy(feat_hbm.at[sidx1], rows1, gsem1))

        def ifetch(j, buf):
            pltpu.async_copy(srcf_hbm.at[pl.ds(base + j * C, C)],
                             sidx[buf], isems[buf])
            pltpu.async_copy(dstf_hbm.at[pl.ds(base + j * C, C)],
                             didx[buf], isems[buf])

        def iwait(buf):
            pltpu.make_async_copy(
                srcf_hbm.at[pl.ds(0, C)], sidx[buf], isems[buf]).wait()
            pltpu.make_async_copy(
                dstf_hbm.at[pl.ds(0, C)], didx[buf], isems[buf]).wait()

        def scatter(buf):
            pltpu.sync_copy(rows[buf], acc.at[didx[buf]], add=True)
            for kk in range(C // L):
                dvec = didx[buf][pl.ds(kk * L, L)]
                plsc.addupdate_scatter(degl, [dvec], ones_vec)

        ifetch(0, 0)
        iwait(0)
        gath[0].start()
        ifetch(1, 1)

        @pl.loop(0, CHUNKS, step=2)
        def _(j):
            iwait(1)
            gath[1].start()
            gath[0].wait()
            scatter(0)

            @pl.when(j + 2 < CHUNKS)
            def _():
                ifetch(j + 2, 0)
                iwait(0)
                gath[0].start()

            gath[1].wait()
            scatter(1)

            @pl.when(j + 3 < CHUNKS)
            def _():
                ifetch(j + 3, 1)

        plsc.subcore_barrier()

        pltpu.sync_copy(acc.at[pl.ds(s * ZROWS, ZROWS)],
                        nsum_hbm.at[c, pl.ds(s * ZROWS, ZROWS)])
        pltpu.sync_copy(degl, deg_hbm.at[pl.ds(w * NDEG, NDEG)])

    return k(feat, srcf, dstf, zrows)


def _tc_deg_recip(degp):
    def body(d_ref, o_ref):
        o_ref[...] = 1.0 / jnp.maximum(jnp.sum(d_ref[...], axis=0), 1.0)

    return pl.pallas_call(
        body,
        out_shape=jax.ShapeDtypeStruct((NDEG // D, D), jnp.float32),
    )(degp)


def _tc_combine(feat, nsum, recip, ws_t, wn_t):
    def body(feat_ref, n_ref, r_ref, ws_ref, wn_ref, o_ref):
        h = (n_ref[0] + n_ref[1]) * r_ref[...]
        o_ref[...] = (
            jnp.dot(feat_ref[...], ws_ref[...],
                    preferred_element_type=jnp.float32)
            + jnp.dot(h, wn_ref[...], preferred_element_type=jnp.float32))

    return pl.pallas_call(
        body,
        grid=(N // R,),
        in_specs=[
            pl.BlockSpec((R, D), lambda i: (i, 0)),
            pl.BlockSpec((NC, R, D), lambda i: (0, i, 0)),
            pl.BlockSpec((R, 1), lambda i: (i, 0)),
            pl.BlockSpec((D, D), lambda i: (0, 0)),
            pl.BlockSpec((D, D), lambda i: (0, 0)),
        ],
        out_specs=pl.BlockSpec((R, D), lambda i: (i, 0)),
        out_shape=jax.ShapeDtypeStruct((N, D), jnp.float32),
    )(feat, nsum, recip, ws_t, wn_t)


def kernel(feat, edge_index, W_self, W_neigh):
    src = edge_index[0]
    dst = edge_index[1]
    pad = EP - E
    srcf = jnp.concatenate([src, jnp.zeros((pad,), jnp.int32)])
    dstf = jnp.concatenate([dst, jnp.full((pad,), DUMMY, jnp.int32)])
    zrows = jnp.zeros((ZROWS, D), jnp.float32)
    nsum, degflat = _sc_segment_sums(feat, srcf, dstf, zrows)
    recip = _tc_deg_recip(degflat.reshape(NW, NDEG // D, D))
    recip_col = recip.reshape(NDEG)[:N][:, None]
    return _tc_combine(feat, nsum, recip_col, W_self.T, W_neigh.T)


# speedup vs baseline: 10.2473x; 2.6493x over previous
import dataclasses
import functools

import jax
import jax.numpy as jnp
from jax import lax
from jax.experimental import pallas as pl
from jax.experimental.pallas import tpu as pltpu
from jax.experimental.pallas import tpu_sc as plsc

N = 10000
E = 320000
D = 128

NC = 2
NS = 16
NW = NC * NS
L = 16

C = 64
CHUNKS = 160
EPW = C * CHUNKS
EP = EPW * NW

NPAD = 10112
ZROWS = NPAD // NS
DUMMY = N
NDEG = 10240

R = 1000


def _sc_segment_sums(feat, srcf, dstf, zrows):
    mesh = plsc.VectorSubcoreMesh(core_axis_name="c", subcore_axis_name="s")
    cp = pltpu.CompilerParams()
    if "needs_layout_passes" in pltpu.CompilerParams.__dataclass_fields__:
        cp = dataclasses.replace(cp, needs_layout_passes=False)

    @functools.partial(
        pl.kernel,
        compiler_params=cp,
        out_type=(
            jax.ShapeDtypeStruct((NC, NPAD, D), jnp.float32),
            jax.ShapeDtypeStruct((NW * NDEG,), jnp.float32),
        ),
        mesh=mesh,
        scratch_types=[
            pltpu.VMEM_SHARED((NPAD, D), jnp.float32),
            pltpu.VMEM((C,), jnp.int32),
            pltpu.VMEM((C,), jnp.int32),
            pltpu.VMEM((C,), jnp.int32),
            pltpu.VMEM((C,), jnp.int32),
            pltpu.VMEM((C, D), jnp.float32),
            pltpu.VMEM((C, D), jnp.float32),
            pltpu.VMEM((NDEG,), jnp.float32),
            pltpu.SemaphoreType.DMA,
            pltpu.SemaphoreType.DMA,
            pltpu.SemaphoreType.DMA,
            pltpu.SemaphoreType.DMA,
        ],
    )
    def k(feat_hbm, srcf_hbm, dstf_hbm, zrows_hbm, nsum_hbm, deg_hbm,
          acc, sidx0, sidx1, didx0, didx1, rows0, rows1, degl,
          gsem0, gsem1, isem0, isem1):
        c = lax.axis_index("c")
        s = lax.axis_index("s")
        w = c * NS + s
        base = w * EPW

        @pl.loop(0, NDEG // L)
        def _(i):
            degl[pl.ds(i * L, L)] = jnp.zeros((L,), jnp.float32)

        pltpu.sync_copy(zrows_hbm, acc.at[pl.ds(s * ZROWS, ZROWS)])
        plsc.subcore_barrier()

        sidx = (sidx0, sidx1)
        didx = (didx0, didx1)
        rows = (rows0, rows1)
        isems = (isem0, isem1)
        ones_vec = jnp.full((L,), 1.0, jnp.float32)

        gath = (pltpu.make_async_copy(feat_hbm.at[sidx0], rows0, gsem0),
                pltpu.make_async_copy(feat_hbm.at[sidx1], rows1, gsem1))

        def ifetch(j, buf):
            pltpu.async_copy(srcf_hbm.at[pl.ds(base + j * C, C)],
                             sidx[buf], isems[buf])
            pltpu.async_copy(dstf_hbm.at[pl.ds(base + j * C, C)],
                             didx[buf], isems[buf])

        def iwait(buf):
            pltpu.make_async_copy(
                srcf_hbm.at[pl.ds(0, C)], sidx[buf], isems[buf]).wait()
            pltpu.make_async_copy(
                dstf_hbm.at[pl.ds(0, C)], didx[buf], isems[buf]).wait()

        def scatter(buf):
            pltpu.sync_copy(rows[buf], acc.at[didx[buf]], add=True)
            for kk in range(C // L):
                dvec = didx[buf][pl.ds(kk * L, L)]
                plsc.addupdate_scatter(degl, [dvec], ones_vec)

        ifetch(0, 0)
        iwait(0)
        gath[0].start()
        ifetch(1, 1)

        @pl.loop(0, CHUNKS, step=2)
        def _(j):
            iwait(1)
            gath[1].start()
            gath[0].wait()
            scatter(0)

            @pl.when(j + 2 < CHUNKS)
            def _():
                ifetch(j + 2, 0)
                iwait(0)
                gath[0].start()

            gath[1].wait()
            scatter(1)

            @pl.when(j + 3 < CHUNKS)
            def _():
                ifetch(j + 3, 1)

        plsc.subcore_barrier()

        pltpu.sync_copy(acc.at[pl.ds(s * ZROWS, ZROWS)],
                        nsum_hbm.at[c, pl.ds(s * ZROWS, ZROWS)])
        pltpu.sync_copy(degl, deg_hbm.at[pl.ds(w * NDEG, NDEG)])

    return k(feat, srcf, dstf, zrows)


def _tc_deg_recip(degp):
    def body(d_ref, o_ref):
        o_ref[...] = 1.0 / jnp.maximum(jnp.sum(d_ref[...], axis=0), 1.0)

    return pl.pallas_call(
        body,
        out_shape=jax.ShapeDtypeStruct((NDEG // D, D), jnp.float32),
    )(degp)


def _tc_combine(feat, nsum, recip, ws_t, wn_t):
    def body(feat_ref, n_ref, r_ref, ws_ref, wn_ref, o_ref):
        h = (n_ref[0] + n_ref[1]) * r_ref[...]
        o_ref[...] = (
            jnp.dot(feat_ref[...], ws_ref[...],
                    preferred_element_type=jnp.float32)
            + jnp.dot(h, wn_ref[...], preferred_element_type=jnp.float32))

    return pl.pallas_call(
        body,
        grid=(N // R,),
        in_specs=[
            pl.BlockSpec((R, D), lambda i: (i, 0)),
            pl.BlockSpec((NC, R, D), lambda i: (0, i, 0)),
            pl.BlockSpec((R, 1), lambda i: (i, 0)),
            pl.BlockSpec((D, D), lambda i: (0, 0)),
            pl.BlockSpec((D, D), lambda i: (0, 0)),
        ],
        out_specs=pl.BlockSpec((R, D), lambda i: (i, 0)),
        out_shape=jax.ShapeDtypeStruct((N, D), jnp.float32),
    )(feat, nsum, recip, ws_t, wn_t)


def kernel(feat, edge_index, W_self, W_neigh):
    src = edge_index[0]
    dst = edge_index[1]
    pad = EP - E
    ppos = jnp.arange(pad, dtype=jnp.int32)
    srcf = jnp.concatenate([src, ppos % N])
    dstf = jnp.concatenate([dst, DUMMY + ppos % (NPAD - N)])
    zrows = jnp.zeros((ZROWS, D), jnp.float32)
    nsum, degflat = _sc_segment_sums(feat, srcf, dstf, zrows)
    recip = _tc_deg_recip(degflat.reshape(NW, NDEG // D, D))
    recip_col = recip.reshape(NDEG)[:N][:, None]
    return _tc_combine(feat, nsum, recip_col, W_self.T, W_neigh.T)


# speedup vs baseline: 12.5999x; 1.2296x over previous
import dataclasses
import functools

import jax
import jax.numpy as jnp
from jax import lax
from jax.experimental import pallas as pl
from jax.experimental.pallas import tpu as pltpu
from jax.experimental.pallas import tpu_sc as plsc

N = 10000
E = 320000
D = 128

NC = 2
NS = 16
NW = NC * NS
L = 16

C = 128
CHUNKS = 80
EPW = C * CHUNKS
EP = EPW * NW

NPAD = 10112
ZROWS = NPAD // NS
DUMMY = N
NDEG = 10240

R = 1000


def _sc_segment_sums(feat, srcf, dstf, zrows):
    mesh = plsc.VectorSubcoreMesh(core_axis_name="c", subcore_axis_name="s")
    cp = pltpu.CompilerParams()
    if "needs_layout_passes" in pltpu.CompilerParams.__dataclass_fields__:
        cp = dataclasses.replace(cp, needs_layout_passes=False)

    @functools.partial(
        pl.kernel,
        compiler_params=cp,
        out_type=(
            jax.ShapeDtypeStruct((NC, NPAD, D), jnp.float32),
            jax.ShapeDtypeStruct((NW * NDEG,), jnp.float32),
        ),
        mesh=mesh,
        scratch_types=[
            pltpu.VMEM_SHARED((NPAD, D), jnp.float32),
            pltpu.VMEM((C,), jnp.int32),
            pltpu.VMEM((C,), jnp.int32),
            pltpu.VMEM((C,), jnp.int32),
            pltpu.VMEM((C,), jnp.int32),
            pltpu.VMEM((C, D), jnp.float32),
            pltpu.VMEM((C, D), jnp.float32),
            pltpu.VMEM((NDEG,), jnp.float32),
            pltpu.SemaphoreType.DMA,
            pltpu.SemaphoreType.DMA,
            pltpu.SemaphoreType.DMA,
            pltpu.SemaphoreType.DMA,
        ],
    )
    def k(feat_hbm, srcf_hbm, dstf_hbm, zrows_hbm, nsum_hbm, deg_hbm,
          acc, sidx0, sidx1, didx0, didx1, rows0, rows1, degl,
          gsem0, gsem1, isem0, isem1):
        c = lax.axis_index("c")
        s = lax.axis_index("s")
        w = c * NS + s
        base = w * EPW

        @pl.loop(0, NDEG // L)
        def _(i):
            degl[pl.ds(i * L, L)] = jnp.zeros((L,), jnp.float32)

        pltpu.sync_copy(zrows_hbm, acc.at[pl.ds(s * ZROWS, ZROWS)])
        plsc.subcore_barrier()

        sidx = (sidx0, sidx1)
        didx = (didx0, didx1)
        rows = (rows0, rows1)
        isems = (isem0, isem1)
        ones_vec = jnp.full((L,), 1.0, jnp.float32)

        gath = (pltpu.make_async_copy(feat_hbm.at[sidx0], rows0, gsem0),
                pltpu.make_async_copy(feat_hbm.at[sidx1], rows1, gsem1))

        def ifetch(j, buf):
            pltpu.async_copy(srcf_hbm.at[pl.ds(base + j * C, C)],
                             sidx[buf], isems[buf])
            pltpu.async_copy(dstf_hbm.at[pl.ds(base + j * C, C)],
                             didx[buf], isems[buf])

        def iwait(buf):
            pltpu.make_async_copy(
                srcf_hbm.at[pl.ds(0, C)], sidx[buf], isems[buf]).wait()
            pltpu.make_async_copy(
                dstf_hbm.at[pl.ds(0, C)], didx[buf], isems[buf]).wait()

        def scatter(buf):
            pltpu.sync_copy(rows[buf], acc.at[didx[buf]], add=True)
            for kk in range(C // L):
                dvec = didx[buf][pl.ds(kk * L, L)]
                plsc.addupdate_scatter(degl, [dvec], ones_vec)

        ifetch(0, 0)
        iwait(0)
        gath[0].start()
        ifetch(1, 1)

        @pl.loop(0, CHUNKS, step=2)
        def _(j):
            iwait(1)
            gath[1].start()
            gath[0].wait()
            scatter(0)

            @pl.when(j + 2 < CHUNKS)
            def _():
                ifetch(j + 2, 0)
                iwait(0)
                gath[0].start()

            gath[1].wait()
            scatter(1)

            @pl.when(j + 3 < CHUNKS)
            def _():
                ifetch(j + 3, 1)

        plsc.subcore_barrier()

        pltpu.sync_copy(acc.at[pl.ds(s * ZROWS, ZROWS)],
                        nsum_hbm.at[c, pl.ds(s * ZROWS, ZROWS)])
        pltpu.sync_copy(degl, deg_hbm.at[pl.ds(w * NDEG, NDEG)])

    return k(feat, srcf, dstf, zrows)


def _tc_deg_recip(degp):
    def body(d_ref, o_ref):
        o_ref[...] = 1.0 / jnp.maximum(jnp.sum(d_ref[...], axis=0), 1.0)

    return pl.pallas_call(
        body,
        out_shape=jax.ShapeDtypeStruct((NDEG // D, D), jnp.float32),
    )(degp)


def _tc_combine(feat, nsum, recip, ws_t, wn_t):
    def body(feat_ref, n_ref, r_ref, ws_ref, wn_ref, o_ref):
        h = (n_ref[0] + n_ref[1]) * r_ref[...]
        o_ref[...] = (
            jnp.dot(feat_ref[...], ws_ref[...],
                    preferred_element_type=jnp.float32)
            + jnp.dot(h, wn_ref[...], preferred_element_type=jnp.float32))

    return pl.pallas_call(
        body,
        grid=(N // R,),
        in_specs=[
            pl.BlockSpec((R, D), lambda i: (i, 0)),
            pl.BlockSpec((NC, R, D), lambda i: (0, i, 0)),
            pl.BlockSpec((R, 1), lambda i: (i, 0)),
            pl.BlockSpec((D, D), lambda i: (0, 0)),
            pl.BlockSpec((D, D), lambda i: (0, 0)),
        ],
        out_specs=pl.BlockSpec((R, D), lambda i: (i, 0)),
        out_shape=jax.ShapeDtypeStruct((N, D), jnp.float32),
    )(feat, nsum, recip, ws_t, wn_t)


def kernel(feat, edge_index, W_self, W_neigh):
    src = edge_index[0]
    dst = edge_index[1]
    pad = EP - E
    ppos = jnp.arange(pad, dtype=jnp.int32)
    srcf = jnp.concatenate([src, ppos % N])
    dstf = jnp.concatenate([dst, DUMMY + ppos % (NPAD - N)])
    zrows = jnp.zeros((ZROWS, D), jnp.float32)
    nsum, degflat = _sc_segment_sums(feat, srcf, dstf, zrows)
    recip = _tc_deg_recip(degflat.reshape(NW, NDEG // D, D))
    recip_col = recip.reshape(NDEG)[:N][:, None]
    return _tc_combine(feat, nsum, recip_col, W_self.T, W_neigh.T)


# speedup vs baseline: 13.7687x; 1.0928x over previous
import dataclasses
import functools

import jax
import jax.numpy as jnp
from jax import lax
from jax.experimental import pallas as pl
from jax.experimental.pallas import tpu as pltpu
from jax.experimental.pallas import tpu_sc as plsc

N = 10000
E = 320000
D = 128

NC = 2
NS = 16
NW = NC * NS
L = 16

C = 64
CHUNKS = 160
EPW = C * CHUNKS
EP = EPW * NW

NPAD = 10112
ZROWS = NPAD // NS
DUMMY = N
NDEG = 10240

R = 1000


def _sc_segment_sums(feat, srcf, dstf, zrows):
    mesh = plsc.VectorSubcoreMesh(core_axis_name="c", subcore_axis_name="s")
    cp = pltpu.CompilerParams()
    if "needs_layout_passes" in pltpu.CompilerParams.__dataclass_fields__:
        cp = dataclasses.replace(cp, needs_layout_passes=False)

    @functools.partial(
        pl.kernel,
        compiler_params=cp,
        out_type=(
            jax.ShapeDtypeStruct((NC, NPAD, D), jnp.float32),
            jax.ShapeDtypeStruct((NW * NDEG,), jnp.float32),
        ),
        mesh=mesh,
        scratch_types=[
            pltpu.VMEM_SHARED((NPAD, D), jnp.float32),
            [pltpu.VMEM((C,), jnp.int32) for _ in range(4)],
            [pltpu.VMEM((C,), jnp.int32) for _ in range(4)],
            [pltpu.VMEM((C, D), jnp.float32) for _ in range(4)],
            pltpu.VMEM((NDEG,), jnp.float32),
            [pltpu.SemaphoreType.DMA for _ in range(4)],
            [pltpu.SemaphoreType.DMA for _ in range(4)],
            [pltpu.SemaphoreType.DMA for _ in range(4)],
        ],
    )
    def k(feat_hbm, srcf_hbm, dstf_hbm, zrows_hbm, nsum_hbm, deg_hbm,
          acc, sidx, didx, rows, degl, gsems, ssems, isems):
        c = lax.axis_index("c")
        s = lax.axis_index("s")
        w = c * NS + s
        base = w * EPW

        @pl.loop(0, NDEG // L)
        def _(i):
            degl[pl.ds(i * L, L)] = jnp.zeros((L,), jnp.float32)

        pltpu.sync_copy(zrows_hbm, acc.at[pl.ds(s * ZROWS, ZROWS)])
        plsc.subcore_barrier()

        ones_vec = jnp.full((L,), 1.0, jnp.float32)

        gath = tuple(pltpu.make_async_copy(feat_hbm.at[sidx[b]], rows[b],
                                           gsems[b]) for b in range(4))
        sct = tuple(pltpu.make_async_copy(rows[b], acc.at[didx[b]],
                                          ssems[b]) for b in range(4))

        def ifetch(j, b):
            pltpu.async_copy(srcf_hbm.at[pl.ds(base + j * C, C)],
                             sidx[b], isems[b])
            pltpu.async_copy(dstf_hbm.at[pl.ds(base + j * C, C)],
                             didx[b], isems[b])

        def iwait(b):
            pltpu.make_async_copy(
                srcf_hbm.at[pl.ds(0, C)], sidx[b], isems[b]).wait()
            pltpu.make_async_copy(
                dstf_hbm.at[pl.ds(0, C)], didx[b], isems[b]).wait()

        def deg(b):
            for kk in range(C // L):
                dvec = didx[b][pl.ds(kk * L, L)]
                plsc.addupdate_scatter(degl, [dvec], ones_vec)

        ifetch(0, 0)
        ifetch(1, 1)
        iwait(0)
        gath[0].start()
        ifetch(2, 2)
        iwait(1)
        gath[1].start()
        gath[0].wait()
        sct[0].start(add=True)
        deg(0)
        ifetch(3, 3)
        iwait(2)
        gath[2].start()
        gath[1].wait()
        sct[1].start(add=True)
        deg(1)

        @pl.loop(2, CHUNKS - 2, step=4)
        def _(j):
            for kk in range(4):
                b = (2 + kk) % 4
                bp1 = (b + 1) % 4
                bm2 = (b + 2) % 4
                sct[bm2].wait()
                ifetch(j + kk + 2, bm2)
                iwait(bp1)
                gath[bp1].start()
                gath[b].wait()
                sct[b].start(add=True)
                deg(b)

        sct[0].wait()
        iwait(3)
        gath[3].start()
        gath[2].wait()
        sct[2].start(add=True)
        deg(2)
        sct[1].wait()
        gath[3].wait()
        sct[3].start(add=True)
        deg(3)
        sct[2].wait()
        sct[3].wait()

        plsc.subcore_barrier()

        pltpu.sync_copy(acc.at[pl.ds(s * ZROWS, ZROWS)],
                        nsum_hbm.at[c, pl.ds(s * ZROWS, ZROWS)])
        pltpu.sync_copy(degl, deg_hbm.at[pl.ds(w * NDEG, NDEG)])

    return k(feat, srcf, dstf, zrows)


def _tc_deg_recip(degp):
    def body(d_ref, o_ref):
        o_ref[...] = 1.0 / jnp.maximum(jnp.sum(d_ref[...], axis=0), 1.0)

    return pl.pallas_call(
        body,
        out_shape=jax.ShapeDtypeStruct((NDEG // D, D), jnp.float32),
    )(degp)


def _tc_combine(feat, nsum, recip, ws_t, wn_t):
    def body(feat_ref, n_ref, r_ref, ws_ref, wn_ref, o_ref):
        h = (n_ref[0] + n_ref[1]) * r_ref[...]
        o_ref[...] = (
            jnp.dot(feat_ref[...], ws_ref[...],
                    preferred_element_type=jnp.float32)
            + jnp.dot(h, wn_ref[...], preferred_element_type=jnp.float32))

    return pl.pallas_call(
        body,
        grid=(N // R,),
        in_specs=[
            pl.BlockSpec((R, D), lambda i: (i, 0)),
            pl.BlockSpec((NC, R, D), lambda i: (0, i, 0)),
            pl.BlockSpec((R, 1), lambda i: (i, 0)),
            pl.BlockSpec((D, D), lambda i: (0, 0)),
            pl.BlockSpec((D, D), lambda i: (0, 0)),
        ],
        out_specs=pl.BlockSpec((R, D), lambda i: (i, 0)),
        out_shape=jax.ShapeDtypeStruct((N, D), jnp.float32),
    )(feat, nsum, recip, ws_t, wn_t)


def kernel(feat, edge_index, W_self, W_neigh):
    src = edge_index[0]
    dst = edge_index[1]
    pad = EP - E
    ppos = jnp.arange(pad, dtype=jnp.int32)
    srcf = jnp.concatenate([src, ppos % N])
    dstf = jnp.concatenate([dst, DUMMY + ppos % (NPAD - N)])
    zrows = jnp.zeros((ZROWS, D), jnp.float32)
    nsum, degflat = _sc_segment_sums(feat, srcf, dstf, zrows)
    recip = _tc_deg_recip(degflat.reshape(NW, NDEG // D, D))
    recip_col = recip.reshape(NDEG)[:N][:, None]
    return _tc_combine(feat, nsum, recip_col, W_self.T, W_neigh.T)
